# Initial kernel scaffold; baseline (speedup 1.0000x reference)
import jax
import jax.numpy as jnp
from jax.experimental import pallas as pl


def kernel(node_type, pos, edge_index, node_embed, Wee, bee, We1, be1, Wg1, bg1, We2, Wg2, Wer, Wn1, bn1, Wgn1, bgn1, Wn2, Wgn2, Wnr, Wf1, bf1, Wf2, bf2, Wf3, bf3):
    raise NotImplementedError("write your pallas kernel here")


# speedup vs baseline: 1.9913x; 1.9913x over previous
import functools

import jax
import jax.numpy as jnp
from jax import lax
from jax.experimental import pallas as pl
from jax.experimental.pallas import tpu as pltpu
from jax.experimental.pallas import tpu_sc as plsc

N = 50000
E = 800000
D = 64
NBLOCKS = 3
CUTOFF = 5.0
DEG = 9

N_PAD = 50048
CHUNK = 128
NW = 32
HALF = D // 2
TE = 3200
TN = 400

_mesh = functools.partial(
    plsc.VectorSubcoreMesh, core_axis_name="c", subcore_axis_name="s")


def _sc_gather(table, idx, d_t):
    m = idx.shape[0]
    nchunks = m // CHUNK
    iters = (nchunks + NW - 1) // NW

    @functools.partial(
        pl.kernel,
        mesh=_mesh(),
        compiler_params=pltpu.CompilerParams(use_tc_tiling_on_sc=False),
        out_type=jax.ShapeDtypeStruct((m, d_t), jnp.float32),
        scratch_types=[
            pltpu.VMEM((CHUNK,), jnp.int32),
            pltpu.VMEM((CHUNK, d_t), jnp.float32),
            pltpu.SemaphoreType.DMA,
        ],
    )
    def k(table_hbm, idx_hbm, out_hbm, idx_v, rows_v, sem):
        wid = lax.axis_index("s") * 2 + lax.axis_index("c")

        def body(i, carry):
            j = i * NW + wid

            @pl.when(j < nchunks)
            def _():
                off = j * CHUNK
                pltpu.sync_copy(idx_hbm.at[pl.ds(off, CHUNK)], idx_v)
                pltpu.async_copy(table_hbm.at[idx_v], rows_v, sem).wait()
                pltpu.sync_copy(rows_v, out_hbm.at[pl.ds(off, CHUNK)])

            return carry

        lax.fori_loop(0, iters, body, 0)

    return k(table, idx)


def _sc_scatter_add(msg, dst, node_feat):
    nchunks = E // CHUNK
    iters = (nchunks + 15) // 16
    rows_per_tile = N_PAD // 16

    @functools.partial(
        pl.kernel,
        mesh=_mesh(),
        compiler_params=pltpu.CompilerParams(use_tc_tiling_on_sc=False),
        out_type=jax.ShapeDtypeStruct((N_PAD, D), jnp.float32),
        scratch_types=[
            pltpu.VMEM((CHUNK,), jnp.int32),
            pltpu.VMEM((CHUNK, HALF), jnp.float32),
            pltpu.VMEM_SHARED((N_PAD, HALF), jnp.float32),
        ],
    )
    def k(msg_hbm, dst_hbm, nf_hbm, out_hbm, idx_v, msg_v, acc_sh):
        c = lax.axis_index("c")
        s = lax.axis_index("s")
        col0 = c * HALF
        row0 = s * rows_per_tile
        pltpu.sync_copy(
            nf_hbm.at[pl.ds(row0, rows_per_tile), pl.ds(col0, HALF)],
            acc_sh.at[pl.ds(row0, rows_per_tile)])
        plsc.subcore_barrier()

        def body(i, carry):
            j = i * 16 + s

            @pl.when(j < nchunks)
            def _():
                off = j * CHUNK
                pltpu.sync_copy(dst_hbm.at[pl.ds(off, CHUNK)], idx_v)
                pltpu.sync_copy(
                    msg_hbm.at[pl.ds(off, CHUNK), pl.ds(col0, HALF)], msg_v)
                pltpu.sync_copy(msg_v, acc_sh.at[idx_v], add=True)

            return carry

        lax.fori_loop(0, iters, body, 0)
        plsc.subcore_barrier()
        pltpu.sync_copy(
            acc_sh.at[pl.ds(row0, rows_per_tile)],
            out_hbm.at[pl.ds(row0, rows_per_tile), pl.ds(col0, HALF)])

    return k(msg, dst, node_feat)


def _swish(x):
    return x * jax.nn.sigmoid(x)


def _tc_edge_init(p2, wee_pad, bee2):
    g_steps = E // TE

    def body(ps_ref, pd_ref, wee_ref, bee_ref, rbf_ref, ef_ref):
        dvec = pd_ref[...] - ps_ref[...]
        d2 = jnp.sum(dvec * dvec, axis=1, keepdims=True)
        bond = jnp.sqrt(d2 + 1e-12)
        r = jnp.maximum(bond, 1e-6)
        col = lax.broadcasted_iota(jnp.int32, (TE, 16), 1).astype(jnp.float32)
        nvec = col + 1.0
        rbf = jnp.sqrt(2.0 / CUTOFF) * jnp.sin(
            nvec * jnp.pi * r / CUTOFF) / r
        ratio = jnp.clip(bond / CUTOFF, 0.0, 1.0)
        env = 1.0 - 6.0 * ratio**5 + 15.0 * ratio**4 - 10.0 * ratio**3
        rbf = jnp.where(col < float(DEG), rbf * env, 0.0)
        rbf_ref[...] = rbf
        pre = jnp.dot(rbf, wee_ref[...],
                      preferred_element_type=jnp.float32) + bee_ref[...]
        ef_ref[...] = _swish(pre)

    return pl.pallas_call(
        body,
        grid=(g_steps,),
        in_specs=[
            pl.BlockSpec((TE, 16), lambda g: (g, 0)),
            pl.BlockSpec((TE, 16), lambda g, _gs=g_steps: (g + _gs, 0)),
            pl.BlockSpec((16, D), lambda g: (0, 0)),
            pl.BlockSpec((1, D), lambda g: (0, 0)),
        ],
        out_specs=[
            pl.BlockSpec((TE, 16), lambda g: (g, 0)),
            pl.BlockSpec((TE, D), lambda g: (g, 0)),
        ],
        out_shape=[
            jax.ShapeDtypeStruct((E, 16), jnp.float32),
            jax.ShapeDtypeStruct((E, D), jnp.float32),
        ],
    )(p2, p2, wee_pad, bee2)


def _tc_block(vivj, ef, rbf, wvi, wvj, wef_eg, wef_ng, beg, bng,
              we2, wg2, wn2, wgn2, wer_pad, wnr_pad):
    g_steps = E // TE

    def body(vi_ref, vj_ref, ef_ref, rbf_ref, wvi_ref, wvj_ref,
             wef_eg_ref, wef_ng_ref, beg_ref, bng_ref,
             we2_ref, wg2_ref, wn2_ref, wgn2_ref, wer_ref, wnr_ref,
             ef2_ref, msg_ref):
        ef0 = ef_ref[...]
        rbf = rbf_ref[...]
        p = (jnp.dot(vi_ref[...], wvi_ref[...],
                     preferred_element_type=jnp.float32)
             + jnp.dot(vj_ref[...], wvj_ref[...],
                       preferred_element_type=jnp.float32))
        pre_eg = p[:, :128] + jnp.dot(
            ef0, wef_eg_ref[...], preferred_element_type=jnp.float32
        ) + beg_ref[...]
        a_eg = _swish(pre_eg)
        h = jnp.dot(a_eg[:, :D], we2_ref[...],
                    preferred_element_type=jnp.float32)
        gate = jax.nn.sigmoid(jnp.dot(a_eg[:, D:], wg2_ref[...],
                                      preferred_element_type=jnp.float32))
        er = jnp.dot(rbf, wer_ref[...], preferred_element_type=jnp.float32)
        ef2 = ef0 + (h * gate) * er
        ef2_ref[...] = ef2
        pre_ng = p[:, 128:] + jnp.dot(
            ef2, wef_ng_ref[...], preferred_element_type=jnp.float32
        ) + bng_ref[...]
        a_ng = _swish(pre_ng)
        h2 = jnp.dot(a_ng[:, :D], wn2_ref[...],
                     preferred_element_type=jnp.float32)
        g2 = jax.nn.sigmoid(jnp.dot(a_ng[:, D:], wgn2_ref[...],
                                    preferred_element_type=jnp.float32))
        nr = jnp.dot(rbf, wnr_ref[...], preferred_element_type=jnp.float32)
        msg_ref[...] = (h2 * g2) * nr

    full = lambda shape: pl.BlockSpec(shape, lambda g: (0, 0))
    return pl.pallas_call(
        body,
        grid=(g_steps,),
        in_specs=[
            pl.BlockSpec((TE, D), lambda g: (g, 0)),
            pl.BlockSpec((TE, D), lambda g, _gs=g_steps: (g + _gs, 0)),
            pl.BlockSpec((TE, D), lambda g: (g, 0)),
            pl.BlockSpec((TE, 16), lambda g: (g, 0)),
            full((D, 256)), full((D, 256)),
            full((D, 128)), full((D, 128)),
            full((1, 128)), full((1, 128)),
            full((D, D)), full((D, D)), full((D, D)), full((D, D)),
            full((16, D)), full((16, D)),
        ],
        out_specs=[
            pl.BlockSpec((TE, D), lambda g: (g, 0)),
            pl.BlockSpec((TE, D), lambda g: (g, 0)),
        ],
        out_shape=[
            jax.ShapeDtypeStruct((E, D), jnp.float32),
            jax.ShapeDtypeStruct((E, D), jnp.float32),
        ],
    )(vivj, vivj, ef, rbf, wvi, wvj, wef_eg, wef_ng, beg, bng,
      we2, wg2, wn2, wgn2, wer_pad, wnr_pad)


def _tc_readout(nf, wf1, bf1, wf2, bf2, wf3, bf3):
    g_steps = N // TN

    def body(nf_ref, wf1_ref, bf1_ref, wf2_ref, bf2_ref, wf3_ref, bf3_ref,
             out_ref, acc_ref):
        g = pl.program_id(0)

        @pl.when(g == 0)
        def _():
            acc_ref[...] = jnp.zeros_like(acc_ref)

        acc_ref[...] += jnp.sum(nf_ref[...], axis=0, keepdims=True)

        @pl.when(g == g_steps - 1)
        def _():
            pooled = acc_ref[...] * (1.0 / N)
            z = _swish(jnp.dot(pooled, wf1_ref[...],
                               preferred_element_type=jnp.float32)
                       + bf1_ref[...])
            z = _swish(jnp.dot(z, wf2_ref[...],
                               preferred_element_type=jnp.float32)
                       + bf2_ref[...])
            out_ref[...] = jnp.dot(z, wf3_ref[...],
                                   preferred_element_type=jnp.float32
                                   ) + bf3_ref[...]

    full = lambda shape: pl.BlockSpec(shape, lambda g: (0, 0))
    return pl.pallas_call(
        body,
        grid=(g_steps,),
        in_specs=[
            pl.BlockSpec((TN, D), lambda g: (g, 0)),
            full((D, D)), full((1, D)), full((D, D)), full((1, D)),
            full((D, 1)), full((1, 1)),
        ],
        out_specs=pl.BlockSpec((1, 1), lambda g: (0, 0)),
        out_shape=jax.ShapeDtypeStruct((1, 1), jnp.float32),
        scratch_shapes=[pltpu.VMEM((1, D), jnp.float32)],
    )(nf, wf1, bf1, wf2, bf2, wf3, bf3)


def kernel(node_type, pos, edge_index, node_embed, Wee, bee, We1, be1,
           Wg1, bg1, We2, Wg2, Wer, Wn1, bn1, Wgn1, bgn1, Wn2, Wgn2, Wnr,
           Wf1, bf1, Wf2, bf2, Wf3, bf3):
    idx2 = edge_index.astype(jnp.int32).reshape(2 * E)
    dst = edge_index[1].astype(jnp.int32)

    pos_pad = jnp.pad(pos, ((0, 0), (0, 13)))
    nt_pad = jnp.pad(node_type.astype(jnp.int32), (0, N_PAD - N))

    p2 = _sc_gather(pos_pad, idx2, 16)
    nf = _sc_gather(node_embed, nt_pad, D)

    wee_pad = jnp.pad(Wee, ((0, 16 - DEG), (0, 0)))
    rbf, ef = _tc_edge_init(p2, wee_pad, bee.reshape(1, D))

    for b in range(NBLOCKS):
        vivj = _sc_gather(nf, idx2, D)
        w1 = jnp.concatenate(
            [We1[b], Wg1[b], Wn1[b], Wgn1[b]], axis=1)
        wvi, wvj, wef = w1[:D], w1[D:2 * D], w1[2 * D:]
        beg = jnp.concatenate([be1[b], bg1[b]]).reshape(1, 128)
        bng = jnp.concatenate([bn1[b], bgn1[b]]).reshape(1, 128)
        wer_pad = jnp.pad(Wer[b], ((0, 16 - DEG), (0, 0)))
        wnr_pad = jnp.pad(Wnr[b], ((0, 16 - DEG), (0, 0)))
        ef, msg = _tc_block(
            vivj, ef, rbf, wvi, wvj, wef[:, :128], wef[:, 128:], beg, bng,
            We2[b], Wg2[b], Wn2[b], Wgn2[b], wer_pad, wnr_pad)
        nf = _sc_scatter_add(msg, dst, nf)

    out = _tc_readout(nf, Wf1, bf1.reshape(1, D), Wf2, bf2.reshape(1, D),
                      Wf3, bf3.reshape(1, 1))
    return out.reshape(1)


# speedup vs baseline: 2.4130x; 1.2118x over previous
import functools

import jax
import jax.numpy as jnp
from jax import lax
from jax.experimental import pallas as pl
from jax.experimental.pallas import tpu as pltpu
from jax.experimental.pallas import tpu_sc as plsc

N = 50000
E = 800000
D = 64
NBLOCKS = 3
CUTOFF = 5.0
DEG = 9

N_PAD = 50048
CHUNK = 128
NW = 32
HALF = D // 2
TE = 3200
TN = 400

_mesh = functools.partial(
    plsc.VectorSubcoreMesh, core_axis_name="c", subcore_axis_name="s")


KBUF = 8


def _sc_gather(table, idx, d_t):
    m = idx.shape[0]
    nchunks = m // CHUNK
    iters = (nchunks + NW * KBUF - 1) // (NW * KBUF)

    @functools.partial(
        pl.kernel,
        mesh=_mesh(),
        compiler_params=pltpu.CompilerParams(use_tc_tiling_on_sc=False),
        out_type=jax.ShapeDtypeStruct((m, d_t), jnp.float32),
        scratch_types=[
            pltpu.VMEM((KBUF, CHUNK), jnp.int32),
            pltpu.VMEM((KBUF, CHUNK, d_t), jnp.float32),
            pltpu.SemaphoreType.DMA,
        ],
    )
    def k(table_hbm, idx_hbm, out_hbm, idx_v, rows_v, sem):
        wid = lax.axis_index("s") * 2 + lax.axis_index("c")

        def body(t, carry):
            def each(fn):
                for kk in range(KBUF):
                    j = (t * KBUF + kk) * NW + wid

                    @pl.when(j < nchunks)
                    def _(j=j, kk=kk):
                        fn(j, kk)

            each(lambda j, kk: pltpu.async_copy(
                idx_hbm.at[pl.ds(j * CHUNK, CHUNK)], idx_v.at[kk], sem))
            each(lambda j, kk: pltpu.make_async_copy(
                idx_hbm.at[pl.ds(j * CHUNK, CHUNK)], idx_v.at[kk],
                sem).wait())
            each(lambda j, kk: pltpu.async_copy(
                table_hbm.at[idx_v.at[kk]], rows_v.at[kk], sem))
            each(lambda j, kk: pltpu.make_async_copy(
                table_hbm.at[idx_v.at[kk]], rows_v.at[kk], sem).wait())
            each(lambda j, kk: pltpu.async_copy(
                rows_v.at[kk], out_hbm.at[pl.ds(j * CHUNK, CHUNK)], sem))
            each(lambda j, kk: pltpu.make_async_copy(
                rows_v.at[kk], out_hbm.at[pl.ds(j * CHUNK, CHUNK)],
                sem).wait())
            return carry

        lax.fori_loop(0, iters, body, 0)

    return k(table, idx)


def _sc_scatter_add(msg, dst, node_feat):
    kb = 4
    nchunks = E // CHUNK
    iters = (nchunks + 16 * kb - 1) // (16 * kb)
    rows_per_tile = N_PAD // 16

    @functools.partial(
        pl.kernel,
        mesh=_mesh(),
        compiler_params=pltpu.CompilerParams(use_tc_tiling_on_sc=False),
        out_type=jax.ShapeDtypeStruct((N_PAD, D), jnp.float32),
        scratch_types=[
            pltpu.VMEM((kb, CHUNK), jnp.int32),
            pltpu.VMEM((kb, CHUNK, HALF), jnp.float32),
            pltpu.VMEM_SHARED((N_PAD, HALF), jnp.float32),
            pltpu.SemaphoreType.DMA,
        ],
    )
    def k(msg_hbm, dst_hbm, nf_hbm, out_hbm, idx_v, msg_v, acc_sh, sem):
        c = lax.axis_index("c")
        s = lax.axis_index("s")
        col0 = c * HALF
        row0 = s * rows_per_tile
        pltpu.sync_copy(
            nf_hbm.at[pl.ds(row0, rows_per_tile), pl.ds(col0, HALF)],
            acc_sh.at[pl.ds(row0, rows_per_tile)])
        plsc.subcore_barrier()

        def body(t, carry):
            def each(fn):
                for kk in range(kb):
                    j = (t * kb + kk) * 16 + s

                    @pl.when(j < nchunks)
                    def _(j=j, kk=kk):
                        fn(j, kk)

            def loads(j, kk):
                off = j * CHUNK
                pltpu.async_copy(
                    dst_hbm.at[pl.ds(off, CHUNK)], idx_v.at[kk], sem)
                pltpu.async_copy(
                    msg_hbm.at[pl.ds(off, CHUNK), pl.ds(col0, HALF)],
                    msg_v.at[kk], sem)

            def load_waits(j, kk):
                off = j * CHUNK
                pltpu.make_async_copy(
                    dst_hbm.at[pl.ds(off, CHUNK)], idx_v.at[kk], sem).wait()
                pltpu.make_async_copy(
                    msg_hbm.at[pl.ds(off, CHUNK), pl.ds(col0, HALF)],
                    msg_v.at[kk], sem).wait()

            each(loads)
            each(load_waits)
            each(lambda j, kk: pltpu.sync_copy(
                msg_v.at[kk], acc_sh.at[idx_v.at[kk]], add=True))
            return carry

        lax.fori_loop(0, iters, body, 0)
        plsc.subcore_barrier()
        pltpu.sync_copy(
            acc_sh.at[pl.ds(row0, rows_per_tile)],
            out_hbm.at[pl.ds(row0, rows_per_tile), pl.ds(col0, HALF)])

    return k(msg, dst, node_feat)


def _swish(x):
    return x * jax.nn.sigmoid(x)


def _tc_edge_init(p2, wee_pad, bee2):
    g_steps = E // TE

    def body(ps_ref, pd_ref, wee_ref, bee_ref, rbf_ref, ef_ref):
        dvec = pd_ref[...] - ps_ref[...]
        d2 = jnp.sum(dvec * dvec, axis=1, keepdims=True)
        bond = jnp.sqrt(d2 + 1e-12)
        r = jnp.maximum(bond, 1e-6)
        col = lax.broadcasted_iota(jnp.int32, (TE, 16), 1).astype(jnp.float32)
        nvec = col + 1.0
        rbf = jnp.sqrt(2.0 / CUTOFF) * jnp.sin(
            nvec * jnp.pi * r / CUTOFF) / r
        ratio = jnp.clip(bond / CUTOFF, 0.0, 1.0)
        env = 1.0 - 6.0 * ratio**5 + 15.0 * ratio**4 - 10.0 * ratio**3
        rbf = jnp.where(col < float(DEG), rbf * env, 0.0)
        rbf_ref[...] = rbf
        pre = jnp.dot(rbf, wee_ref[...],
                      preferred_element_type=jnp.float32) + bee_ref[...]
        ef_ref[...] = _swish(pre)

    return pl.pallas_call(
        body,
        grid=(g_steps,),
        in_specs=[
            pl.BlockSpec((TE, 16), lambda g: (g, 0)),
            pl.BlockSpec((TE, 16), lambda g, _gs=g_steps: (g + _gs, 0)),
            pl.BlockSpec((16, D), lambda g: (0, 0)),
            pl.BlockSpec((1, D), lambda g: (0, 0)),
        ],
        out_specs=[
            pl.BlockSpec((TE, 16), lambda g: (g, 0)),
            pl.BlockSpec((TE, D), lambda g: (g, 0)),
        ],
        out_shape=[
            jax.ShapeDtypeStruct((E, 16), jnp.float32),
            jax.ShapeDtypeStruct((E, D), jnp.float32),
        ],
    )(p2, p2, wee_pad, bee2)


def _tc_block(vivj, ef, rbf, wvi, wvj, wef_eg, wef_ng, beg, bng,
              we2, wg2, wn2, wgn2, wer_pad, wnr_pad):
    g_steps = E // TE

    def body(vi_ref, vj_ref, ef_ref, rbf_ref, wvi_ref, wvj_ref,
             wef_eg_ref, wef_ng_ref, beg_ref, bng_ref,
             we2_ref, wg2_ref, wn2_ref, wgn2_ref, wer_ref, wnr_ref,
             ef2_ref, msg_ref):
        ef0 = ef_ref[...]
        rbf = rbf_ref[...]
        p = (jnp.dot(vi_ref[...], wvi_ref[...],
                     preferred_element_type=jnp.float32)
             + jnp.dot(vj_ref[...], wvj_ref[...],
                       preferred_element_type=jnp.float32))
        pre_eg = p[:, :128] + jnp.dot(
            ef0, wef_eg_ref[...], preferred_element_type=jnp.float32
        ) + beg_ref[...]
        a_eg = _swish(pre_eg)
        h = jnp.dot(a_eg[:, :D], we2_ref[...],
                    preferred_element_type=jnp.float32)
        gate = jax.nn.sigmoid(jnp.dot(a_eg[:, D:], wg2_ref[...],
                                      preferred_element_type=jnp.float32))
        er = jnp.dot(rbf, wer_ref[...], preferred_element_type=jnp.float32)
        ef2 = ef0 + (h * gate) * er
        ef2_ref[...] = ef2
        pre_ng = p[:, 128:] + jnp.dot(
            ef2, wef_ng_ref[...], preferred_element_type=jnp.float32
        ) + bng_ref[...]
        a_ng = _swish(pre_ng)
        h2 = jnp.dot(a_ng[:, :D], wn2_ref[...],
                     preferred_element_type=jnp.float32)
        g2 = jax.nn.sigmoid(jnp.dot(a_ng[:, D:], wgn2_ref[...],
                                    preferred_element_type=jnp.float32))
        nr = jnp.dot(rbf, wnr_ref[...], preferred_element_type=jnp.float32)
        msg_ref[...] = (h2 * g2) * nr

    full = lambda shape: pl.BlockSpec(shape, lambda g: (0, 0))
    return pl.pallas_call(
        body,
        grid=(g_steps,),
        in_specs=[
            pl.BlockSpec((TE, D), lambda g: (g, 0)),
            pl.BlockSpec((TE, D), lambda g, _gs=g_steps: (g + _gs, 0)),
            pl.BlockSpec((TE, D), lambda g: (g, 0)),
            pl.BlockSpec((TE, 16), lambda g: (g, 0)),
            full((D, 256)), full((D, 256)),
            full((D, 128)), full((D, 128)),
            full((1, 128)), full((1, 128)),
            full((D, D)), full((D, D)), full((D, D)), full((D, D)),
            full((16, D)), full((16, D)),
        ],
        out_specs=[
            pl.BlockSpec((TE, D), lambda g: (g, 0)),
            pl.BlockSpec((TE, D), lambda g: (g, 0)),
        ],
        out_shape=[
            jax.ShapeDtypeStruct((E, D), jnp.float32),
            jax.ShapeDtypeStruct((E, D), jnp.float32),
        ],
    )(vivj, vivj, ef, rbf, wvi, wvj, wef_eg, wef_ng, beg, bng,
      we2, wg2, wn2, wgn2, wer_pad, wnr_pad)


def _tc_readout(nf, wf1, bf1, wf2, bf2, wf3, bf3):
    g_steps = N // TN

    def body(nf_ref, wf1_ref, bf1_ref, wf2_ref, bf2_ref, wf3_ref, bf3_ref,
             out_ref, acc_ref):
        g = pl.program_id(0)

        @pl.when(g == 0)
        def _():
            acc_ref[...] = jnp.zeros_like(acc_ref)

        acc_ref[...] += jnp.sum(nf_ref[...], axis=0, keepdims=True)

        @pl.when(g == g_steps - 1)
        def _():
            pooled = acc_ref[...] * (1.0 / N)
            z = _swish(jnp.dot(pooled, wf1_ref[...],
                               preferred_element_type=jnp.float32)
                       + bf1_ref[...])
            z = _swish(jnp.dot(z, wf2_ref[...],
                               preferred_element_type=jnp.float32)
                       + bf2_ref[...])
            out_ref[...] = jnp.dot(z, wf3_ref[...],
                                   preferred_element_type=jnp.float32
                                   ) + bf3_ref[...]

    full = lambda shape: pl.BlockSpec(shape, lambda g: (0, 0))
    return pl.pallas_call(
        body,
        grid=(g_steps,),
        in_specs=[
            pl.BlockSpec((TN, D), lambda g: (g, 0)),
            full((D, D)), full((1, D)), full((D, D)), full((1, D)),
            full((D, 1)), full((1, 1)),
        ],
        out_specs=pl.BlockSpec((1, 1), lambda g: (0, 0)),
        out_shape=jax.ShapeDtypeStruct((1, 1), jnp.float32),
        scratch_shapes=[pltpu.VMEM((1, D), jnp.float32)],
    )(nf, wf1, bf1, wf2, bf2, wf3, bf3)


def kernel(node_type, pos, edge_index, node_embed, Wee, bee, We1, be1,
           Wg1, bg1, We2, Wg2, Wer, Wn1, bn1, Wgn1, bgn1, Wn2, Wgn2, Wnr,
           Wf1, bf1, Wf2, bf2, Wf3, bf3):
    idx2 = edge_index.astype(jnp.int32).reshape(2 * E)
    dst = edge_index[1].astype(jnp.int32)

    pos_pad = jnp.pad(pos, ((0, 0), (0, 13)))
    nt_pad = jnp.pad(node_type.astype(jnp.int32), (0, N_PAD - N))

    p2 = _sc_gather(pos_pad, idx2, 16)
    nf = _sc_gather(node_embed, nt_pad, D)

    wee_pad = jnp.pad(Wee, ((0, 16 - DEG), (0, 0)))
    rbf, ef = _tc_edge_init(p2, wee_pad, bee.reshape(1, D))

    for b in range(NBLOCKS):
        vivj = _sc_gather(nf, idx2, D)
        w1 = jnp.concatenate(
            [We1[b], Wg1[b], Wn1[b], Wgn1[b]], axis=1)
        wvi, wvj, wef = w1[:D], w1[D:2 * D], w1[2 * D:]
        beg = jnp.concatenate([be1[b], bg1[b]]).reshape(1, 128)
        bng = jnp.concatenate([bn1[b], bgn1[b]]).reshape(1, 128)
        wer_pad = jnp.pad(Wer[b], ((0, 16 - DEG), (0, 0)))
        wnr_pad = jnp.pad(Wnr[b], ((0, 16 - DEG), (0, 0)))
        ef, msg = _tc_block(
            vivj, ef, rbf, wvi, wvj, wef[:, :128], wef[:, 128:], beg, bng,
            We2[b], Wg2[b], Wn2[b], Wgn2[b], wer_pad, wnr_pad)
        nf = _sc_scatter_add(msg, dst, nf)

    out = _tc_readout(nf, Wf1, bf1.reshape(1, D), Wf2, bf2.reshape(1, D),
                      Wf3, bf3.reshape(1, 1))
    return out.reshape(1)


# speedup vs baseline: 2.6833x; 1.1120x over previous
import functools

import jax
import jax.numpy as jnp
from jax import lax
from jax.experimental import pallas as pl
from jax.experimental.pallas import tpu as pltpu
from jax.experimental.pallas import tpu_sc as plsc

N = 50000
E = 800000
D = 64
NBLOCKS = 3
CUTOFF = 5.0
DEG = 9

N_PAD = 50048
CHUNK = 128
NW = 32
HALF = D // 2
TE = 3200
TN = 400

_mesh = functools.partial(
    plsc.VectorSubcoreMesh, core_axis_name="c", subcore_axis_name="s")


KBUF = 8


def _sc_gather(table, idx, d_t):
    m = idx.shape[0]
    nchunks = m // CHUNK
    iters = (nchunks + NW * KBUF - 1) // (NW * KBUF)

    @functools.partial(
        pl.kernel,
        mesh=_mesh(),
        compiler_params=pltpu.CompilerParams(use_tc_tiling_on_sc=False),
        out_type=jax.ShapeDtypeStruct((m, d_t), jnp.float32),
        scratch_types=[
            pltpu.VMEM((KBUF, CHUNK), jnp.int32),
            pltpu.VMEM((KBUF, CHUNK, d_t), jnp.float32),
            pltpu.SemaphoreType.DMA,
        ],
    )
    def k(table_hbm, idx_hbm, out_hbm, idx_v, rows_v, sem):
        wid = lax.axis_index("s") * 2 + lax.axis_index("c")

        def body(t, carry):
            def each(fn):
                for kk in range(KBUF):
                    j = (t * KBUF + kk) * NW + wid

                    @pl.when(j < nchunks)
                    def _(j=j, kk=kk):
                        fn(j, kk)

            each(lambda j, kk: pltpu.async_copy(
                idx_hbm.at[pl.ds(j * CHUNK, CHUNK)], idx_v.at[kk], sem))
            each(lambda j, kk: pltpu.make_async_copy(
                idx_hbm.at[pl.ds(j * CHUNK, CHUNK)], idx_v.at[kk],
                sem).wait())
            each(lambda j, kk: pltpu.async_copy(
                table_hbm.at[idx_v.at[kk]], rows_v.at[kk], sem))
            each(lambda j, kk: pltpu.make_async_copy(
                table_hbm.at[idx_v.at[kk]], rows_v.at[kk], sem).wait())
            each(lambda j, kk: pltpu.async_copy(
                rows_v.at[kk], out_hbm.at[pl.ds(j * CHUNK, CHUNK)], sem))
            each(lambda j, kk: pltpu.make_async_copy(
                rows_v.at[kk], out_hbm.at[pl.ds(j * CHUNK, CHUNK)],
                sem).wait())
            return carry

        lax.fori_loop(0, iters, body, 0)

    return k(table, idx)


def _sc_scatter_add(msg, dst, node_feat):
    kb = 4
    nchunks = E // CHUNK
    iters = (nchunks + 16 * kb - 1) // (16 * kb)
    rows_per_tile = N_PAD // 16

    @functools.partial(
        pl.kernel,
        mesh=_mesh(),
        compiler_params=pltpu.CompilerParams(use_tc_tiling_on_sc=False),
        out_type=jax.ShapeDtypeStruct((N_PAD, D), jnp.float32),
        scratch_types=[
            pltpu.VMEM((kb, CHUNK), jnp.int32),
            pltpu.VMEM((kb, CHUNK, HALF), jnp.float32),
            pltpu.VMEM_SHARED((N_PAD, HALF), jnp.float32),
            pltpu.SemaphoreType.DMA,
        ],
    )
    def k(msg_hbm, dst_hbm, nf_hbm, out_hbm, idx_v, msg_v, acc_sh, sem):
        c = lax.axis_index("c")
        s = lax.axis_index("s")
        col0 = c * HALF
        row0 = s * rows_per_tile
        pltpu.sync_copy(
            nf_hbm.at[pl.ds(row0, rows_per_tile), pl.ds(col0, HALF)],
            acc_sh.at[pl.ds(row0, rows_per_tile)])
        plsc.subcore_barrier()

        def body(t, carry):
            def each(fn):
                for kk in range(kb):
                    j = (t * kb + kk) * 16 + s

                    @pl.when(j < nchunks)
                    def _(j=j, kk=kk):
                        fn(j, kk)

            def loads(j, kk):
                off = j * CHUNK
                pltpu.async_copy(
                    dst_hbm.at[pl.ds(off, CHUNK)], idx_v.at[kk], sem)
                pltpu.async_copy(
                    msg_hbm.at[pl.ds(off, CHUNK), pl.ds(col0, HALF)],
                    msg_v.at[kk], sem)

            def load_waits(j, kk):
                off = j * CHUNK
                pltpu.make_async_copy(
                    dst_hbm.at[pl.ds(off, CHUNK)], idx_v.at[kk], sem).wait()
                pltpu.make_async_copy(
                    msg_hbm.at[pl.ds(off, CHUNK), pl.ds(col0, HALF)],
                    msg_v.at[kk], sem).wait()

            each(loads)
            each(load_waits)
            each(lambda j, kk: pltpu.sync_copy(
                msg_v.at[kk], acc_sh.at[idx_v.at[kk]], add=True))
            return carry

        lax.fori_loop(0, iters, body, 0)
        plsc.subcore_barrier()
        pltpu.sync_copy(
            acc_sh.at[pl.ds(row0, rows_per_tile)],
            out_hbm.at[pl.ds(row0, rows_per_tile), pl.ds(col0, HALF)])

    return k(msg, dst, node_feat)


def _swish(x):
    return x * jax.nn.sigmoid(x)


def _bdot(a, b):
    return jnp.dot(a.astype(jnp.bfloat16), b.astype(jnp.bfloat16),
                   preferred_element_type=jnp.float32)


def _tc_edge_init(p2, wee_pad, bee2):
    g_steps = E // TE

    def body(ps_ref, pd_ref, wee_ref, bee_ref, rbf_ref, ef_ref):
        dvec = pd_ref[...] - ps_ref[...]
        d2 = jnp.sum(dvec * dvec, axis=1, keepdims=True)
        bond = jnp.sqrt(d2 + 1e-12)
        r = jnp.maximum(bond, 1e-6)
        col = lax.broadcasted_iota(jnp.int32, (TE, 16), 1).astype(jnp.float32)
        nvec = col + 1.0
        t = nvec * (jnp.pi / CUTOFF) * jnp.minimum(r, CUTOFF)
        k = jnp.floor(t * (1.0 / jnp.pi) + 0.5)
        u = t - k * jnp.pi
        sign = 1.0 - 2.0 * (k - 2.0 * jnp.floor(k * 0.5))
        u2 = u * u
        sin_t = sign * u * (1.0 + u2 * (-1.0 / 6.0 + u2 * (
            1.0 / 120.0 + u2 * (-1.0 / 5040.0 + u2 * (1.0 / 362880.0)))))
        rbf = jnp.sqrt(2.0 / CUTOFF) * sin_t / r
        ratio = jnp.clip(bond / CUTOFF, 0.0, 1.0)
        env = 1.0 - 6.0 * ratio**5 + 15.0 * ratio**4 - 10.0 * ratio**3
        rbf = jnp.where(col < float(DEG), rbf * env, 0.0)
        rbf_ref[...] = rbf
        pre = jnp.dot(rbf, wee_ref[...],
                      preferred_element_type=jnp.float32) + bee_ref[...]
        ef_ref[...] = _swish(pre)

    return pl.pallas_call(
        body,
        grid=(g_steps,),
        in_specs=[
            pl.BlockSpec((TE, 16), lambda g: (g, 0)),
            pl.BlockSpec((TE, 16), lambda g, _gs=g_steps: (g + _gs, 0)),
            pl.BlockSpec((16, D), lambda g: (0, 0)),
            pl.BlockSpec((1, D), lambda g: (0, 0)),
        ],
        out_specs=[
            pl.BlockSpec((TE, 16), lambda g: (g, 0)),
            pl.BlockSpec((TE, D), lambda g: (g, 0)),
        ],
        out_shape=[
            jax.ShapeDtypeStruct((E, 16), jnp.float32),
            jax.ShapeDtypeStruct((E, D), jnp.float32),
        ],
    )(p2, p2, wee_pad, bee2)


def _tc_block(vivj, ef, rbf, wvi, wvj, wef_eg, wef_ng, beg, bng,
              we2, wg2, wn2, wgn2, wer_pad, wnr_pad):
    g_steps = E // TE

    def body(vi_ref, vj_ref, ef_ref, rbf_ref, wvi_ref, wvj_ref,
             wef_eg_ref, wef_ng_ref, beg_ref, bng_ref,
             we2_ref, wg2_ref, wn2_ref, wgn2_ref, wer_ref, wnr_ref,
             ef2_ref, msg_ref):
        ef0 = ef_ref[...]
        rbf = rbf_ref[...]
        p = (_bdot(vi_ref[...], wvi_ref[...])
             + _bdot(vj_ref[...], wvj_ref[...]))
        pre_eg = p[:, :128] + _bdot(ef0, wef_eg_ref[...]) + beg_ref[...]
        a_eg = _swish(pre_eg)
        h = _bdot(a_eg[:, :D], we2_ref[...])
        gate = jax.nn.sigmoid(_bdot(a_eg[:, D:], wg2_ref[...]))
        er = _bdot(rbf, wer_ref[...])
        ef2 = ef0 + (h * gate) * er
        ef2_ref[...] = ef2
        pre_ng = p[:, 128:] + _bdot(ef2, wef_ng_ref[...]) + bng_ref[...]
        a_ng = _swish(pre_ng)
        h2 = _bdot(a_ng[:, :D], wn2_ref[...])
        g2 = jax.nn.sigmoid(_bdot(a_ng[:, D:], wgn2_ref[...]))
        nr = _bdot(rbf, wnr_ref[...])
        msg_ref[...] = (h2 * g2) * nr

    full = lambda shape: pl.BlockSpec(shape, lambda g: (0, 0))
    return pl.pallas_call(
        body,
        grid=(g_steps,),
        in_specs=[
            pl.BlockSpec((TE, D), lambda g: (g, 0)),
            pl.BlockSpec((TE, D), lambda g, _gs=g_steps: (g + _gs, 0)),
            pl.BlockSpec((TE, D), lambda g: (g, 0)),
            pl.BlockSpec((TE, 16), lambda g: (g, 0)),
            full((D, 256)), full((D, 256)),
            full((D, 128)), full((D, 128)),
            full((1, 128)), full((1, 128)),
            full((D, D)), full((D, D)), full((D, D)), full((D, D)),
            full((16, D)), full((16, D)),
        ],
        out_specs=[
            pl.BlockSpec((TE, D), lambda g: (g, 0)),
            pl.BlockSpec((TE, D), lambda g: (g, 0)),
        ],
        out_shape=[
            jax.ShapeDtypeStruct((E, D), jnp.float32),
            jax.ShapeDtypeStruct((E, D), jnp.float32),
        ],
    )(vivj, vivj, ef, rbf, wvi, wvj, wef_eg, wef_ng, beg, bng,
      we2, wg2, wn2, wgn2, wer_pad, wnr_pad)


def _tc_readout(nf, wf1, bf1, wf2, bf2, wf3, bf3):
    g_steps = N // TN

    def body(nf_ref, wf1_ref, bf1_ref, wf2_ref, bf2_ref, wf3_ref, bf3_ref,
             out_ref, acc_ref):
        g = pl.program_id(0)

        @pl.when(g == 0)
        def _():
            acc_ref[...] = jnp.zeros_like(acc_ref)

        acc_ref[...] += jnp.sum(nf_ref[...], axis=0, keepdims=True)

        @pl.when(g == g_steps - 1)
        def _():
            pooled = acc_ref[...] * (1.0 / N)
            z = _swish(jnp.dot(pooled, wf1_ref[...],
                               preferred_element_type=jnp.float32)
                       + bf1_ref[...])
            z = _swish(jnp.dot(z, wf2_ref[...],
                               preferred_element_type=jnp.float32)
                       + bf2_ref[...])
            out_ref[...] = jnp.dot(z, wf3_ref[...],
                                   preferred_element_type=jnp.float32
                                   ) + bf3_ref[...]

    full = lambda shape: pl.BlockSpec(shape, lambda g: (0, 0))
    return pl.pallas_call(
        body,
        grid=(g_steps,),
        in_specs=[
            pl.BlockSpec((TN, D), lambda g: (g, 0)),
            full((D, D)), full((1, D)), full((D, D)), full((1, D)),
            full((D, 1)), full((1, 1)),
        ],
        out_specs=pl.BlockSpec((1, 1), lambda g: (0, 0)),
        out_shape=jax.ShapeDtypeStruct((1, 1), jnp.float32),
        scratch_shapes=[pltpu.VMEM((1, D), jnp.float32)],
    )(nf, wf1, bf1, wf2, bf2, wf3, bf3)


def kernel(node_type, pos, edge_index, node_embed, Wee, bee, We1, be1,
           Wg1, bg1, We2, Wg2, Wer, Wn1, bn1, Wgn1, bgn1, Wn2, Wgn2, Wnr,
           Wf1, bf1, Wf2, bf2, Wf3, bf3):
    idx2 = edge_index.astype(jnp.int32).reshape(2 * E)
    dst = edge_index[1].astype(jnp.int32)

    pos_pad = jnp.pad(pos, ((0, 0), (0, 13)))
    nt_pad = jnp.pad(node_type.astype(jnp.int32), (0, N_PAD - N))

    p2 = _sc_gather(pos_pad, idx2, 16)
    nf = _sc_gather(node_embed, nt_pad, D)

    wee_pad = jnp.pad(Wee, ((0, 16 - DEG), (0, 0)))
    rbf, ef = _tc_edge_init(p2, wee_pad, bee.reshape(1, D))

    for b in range(NBLOCKS):
        vivj = _sc_gather(nf, idx2, D)
        w1 = jnp.concatenate(
            [We1[b], Wg1[b], Wn1[b], Wgn1[b]], axis=1)
        wvi, wvj, wef = w1[:D], w1[D:2 * D], w1[2 * D:]
        beg = jnp.concatenate([be1[b], bg1[b]]).reshape(1, 128)
        bng = jnp.concatenate([bn1[b], bgn1[b]]).reshape(1, 128)
        wer_pad = jnp.pad(Wer[b], ((0, 16 - DEG), (0, 0)))
        wnr_pad = jnp.pad(Wnr[b], ((0, 16 - DEG), (0, 0)))
        ef, msg = _tc_block(
            vivj, ef, rbf, wvi, wvj, wef[:, :128], wef[:, 128:], beg, bng,
            We2[b], Wg2[b], Wn2[b], Wgn2[b], wer_pad, wnr_pad)
        nf = _sc_scatter_add(msg, dst, nf)

    out = _tc_readout(nf, Wf1, bf1.reshape(1, D), Wf2, bf2.reshape(1, D),
                      Wf3, bf3.reshape(1, 1))
    return out.reshape(1)


# speedup vs baseline: 2.7154x; 1.0120x over previous
import functools

import jax
import jax.numpy as jnp
from jax import lax
from jax.experimental import pallas as pl
from jax.experimental.pallas import tpu as pltpu
from jax.experimental.pallas import tpu_sc as plsc

N = 50000
E = 800000
D = 64
NBLOCKS = 3
CUTOFF = 5.0
DEG = 9

N_PAD = 50048
CHUNK = 128
NW = 32
HALF = D // 2
TE = 3200
TN = 400

_mesh = functools.partial(
    plsc.VectorSubcoreMesh, core_axis_name="c", subcore_axis_name="s")


KBUF = 8


def _gather_loop(table_hbm, idx_hbm, out_hbm, idx_v, rows_v, sem, nchunks,
                 wid):
    iters = (nchunks + NW * KBUF - 1) // (NW * KBUF)

    def body(t, carry):
        def each(fn):
            for kk in range(KBUF):
                j = (t * KBUF + kk) * NW + wid

                @pl.when(j < nchunks)
                def _(j=j, kk=kk):
                    fn(j, kk)

        each(lambda j, kk: pltpu.async_copy(
            idx_hbm.at[pl.ds(j * CHUNK, CHUNK)], idx_v.at[kk], sem))
        each(lambda j, kk: pltpu.make_async_copy(
            idx_hbm.at[pl.ds(j * CHUNK, CHUNK)], idx_v.at[kk],
            sem).wait())
        each(lambda j, kk: pltpu.async_copy(
            table_hbm.at[idx_v.at[kk]], rows_v.at[kk], sem))
        each(lambda j, kk: pltpu.make_async_copy(
            table_hbm.at[idx_v.at[kk]], rows_v.at[kk], sem).wait())
        each(lambda j, kk: pltpu.async_copy(
            rows_v.at[kk], out_hbm.at[pl.ds(j * CHUNK, CHUNK)], sem))
        each(lambda j, kk: pltpu.make_async_copy(
            rows_v.at[kk], out_hbm.at[pl.ds(j * CHUNK, CHUNK)],
            sem).wait())
        return carry

    lax.fori_loop(0, iters, body, 0)


def _sc_gather(table, idx, d_t):
    m = idx.shape[0]
    nchunks = m // CHUNK

    @functools.partial(
        pl.kernel,
        mesh=_mesh(),
        compiler_params=pltpu.CompilerParams(use_tc_tiling_on_sc=False),
        out_type=jax.ShapeDtypeStruct((m, d_t), jnp.float32),
        scratch_types=[
            pltpu.VMEM((KBUF, CHUNK), jnp.int32),
            pltpu.VMEM((KBUF, CHUNK, d_t), jnp.float32),
            pltpu.SemaphoreType.DMA,
        ],
    )
    def k(table_hbm, idx_hbm, out_hbm, idx_v, rows_v, sem):
        wid = lax.axis_index("s") * 2 + lax.axis_index("c")
        _gather_loop(table_hbm, idx_hbm, out_hbm, idx_v, rows_v, sem,
                     nchunks, wid)

    return k(table, idx)


def _sc_gather2(pos_pad, idx2, node_embed, nt_pad):
    @functools.partial(
        pl.kernel,
        mesh=_mesh(),
        compiler_params=pltpu.CompilerParams(use_tc_tiling_on_sc=False),
        out_type=[
            jax.ShapeDtypeStruct((2 * E, 16), jnp.float32),
            jax.ShapeDtypeStruct((N_PAD, D), jnp.float32),
        ],
        scratch_types=[
            pltpu.VMEM((KBUF, CHUNK), jnp.int32),
            pltpu.VMEM((KBUF, CHUNK, 16), jnp.float32),
            pltpu.VMEM((KBUF, CHUNK, D), jnp.float32),
            pltpu.SemaphoreType.DMA,
        ],
    )
    def k(pos_hbm, idx2_hbm, emb_hbm, nt_hbm, p2_out, nf_out,
          idx_v, rows16_v, rows64_v, sem):
        wid = lax.axis_index("s") * 2 + lax.axis_index("c")
        _gather_loop(pos_hbm, idx2_hbm, p2_out, idx_v, rows16_v, sem,
                     (2 * E) // CHUNK, wid)
        _gather_loop(emb_hbm, nt_hbm, nf_out, idx_v, rows64_v, sem,
                     N_PAD // CHUNK, wid)

    return k(pos_pad, idx2, node_embed, nt_pad)


def _sc_scatter_add(msg, dst, node_feat):
    kb = 4
    nchunks = E // CHUNK
    iters = (nchunks + 16 * kb - 1) // (16 * kb)
    rows_per_tile = N_PAD // 16

    @functools.partial(
        pl.kernel,
        mesh=_mesh(),
        compiler_params=pltpu.CompilerParams(use_tc_tiling_on_sc=False),
        out_type=jax.ShapeDtypeStruct((N_PAD, D), jnp.float32),
        scratch_types=[
            pltpu.VMEM((kb, CHUNK), jnp.int32),
            pltpu.VMEM((kb, CHUNK, HALF), jnp.float32),
            pltpu.VMEM_SHARED((N_PAD, HALF), jnp.float32),
            pltpu.SemaphoreType.DMA,
        ],
    )
    def k(msg_hbm, dst_hbm, nf_hbm, out_hbm, idx_v, msg_v, acc_sh, sem):
        c = lax.axis_index("c")
        s = lax.axis_index("s")
        col0 = c * HALF
        row0 = s * rows_per_tile
        pltpu.sync_copy(
            nf_hbm.at[pl.ds(row0, rows_per_tile), pl.ds(col0, HALF)],
            acc_sh.at[pl.ds(row0, rows_per_tile)])
        plsc.subcore_barrier()

        def body(t, carry):
            def each(fn):
                for kk in range(kb):
                    j = (t * kb + kk) * 16 + s

                    @pl.when(j < nchunks)
                    def _(j=j, kk=kk):
                        fn(j, kk)

            def loads(j, kk):
                off = j * CHUNK
                pltpu.async_copy(
                    dst_hbm.at[pl.ds(off, CHUNK)], idx_v.at[kk], sem)
                pltpu.async_copy(
                    msg_hbm.at[pl.ds(off, CHUNK), pl.ds(col0, HALF)],
                    msg_v.at[kk], sem)

            def load_waits(j, kk):
                off = j * CHUNK
                pltpu.make_async_copy(
                    dst_hbm.at[pl.ds(off, CHUNK)], idx_v.at[kk], sem).wait()
                pltpu.make_async_copy(
                    msg_hbm.at[pl.ds(off, CHUNK), pl.ds(col0, HALF)],
                    msg_v.at[kk], sem).wait()

            each(loads)
            each(load_waits)
            each(lambda j, kk: pltpu.sync_copy(
                msg_v.at[kk], acc_sh.at[idx_v.at[kk]], add=True))
            return carry

        lax.fori_loop(0, iters, body, 0)
        plsc.subcore_barrier()
        pltpu.sync_copy(
            acc_sh.at[pl.ds(row0, rows_per_tile)],
            out_hbm.at[pl.ds(row0, rows_per_tile), pl.ds(col0, HALF)])

    return k(msg, dst, node_feat)


def _swish(x):
    return x * jax.nn.sigmoid(x)


def _bdot(a, b):
    return jnp.dot(a.astype(jnp.bfloat16), b.astype(jnp.bfloat16),
                   preferred_element_type=jnp.float32)


def _rbf_ef(ps, pd, wee, bee):
    dvec = pd - ps
    d2 = jnp.sum(dvec * dvec, axis=1, keepdims=True)
    bond = jnp.sqrt(d2 + 1e-12)
    r = jnp.maximum(bond, 1e-6)
    col = lax.broadcasted_iota(jnp.int32, (TE, 16), 1).astype(jnp.float32)
    nvec = col + 1.0
    t = nvec * (jnp.pi / CUTOFF) * jnp.minimum(r, CUTOFF)
    k = jnp.floor(t * (1.0 / jnp.pi) + 0.5)
    u = t - k * jnp.pi
    sign = 1.0 - 2.0 * (k - 2.0 * jnp.floor(k * 0.5))
    u2 = u * u
    sin_t = sign * u * (1.0 + u2 * (-1.0 / 6.0 + u2 * (
        1.0 / 120.0 + u2 * (-1.0 / 5040.0 + u2 * (1.0 / 362880.0)))))
    rbf = jnp.sqrt(2.0 / CUTOFF) * sin_t / r
    ratio = jnp.clip(bond / CUTOFF, 0.0, 1.0)
    env = 1.0 - 6.0 * ratio**5 + 15.0 * ratio**4 - 10.0 * ratio**3
    rbf = jnp.where(col < float(DEG), rbf * env, 0.0)
    pre = jnp.dot(rbf, wee, preferred_element_type=jnp.float32) + bee
    return rbf, _swish(pre)


def _block_math(vi, vj, ef0, rbf, wvi, wvj, wef_eg, wef_ng, beg, bng,
                we2, wg2, wn2, wgn2, wer, wnr):
    p = _bdot(vi, wvi) + _bdot(vj, wvj)
    pre_eg = p[:, :128] + _bdot(ef0, wef_eg) + beg
    a_eg = _swish(pre_eg)
    h = _bdot(a_eg[:, :D], we2)
    gate = jax.nn.sigmoid(_bdot(a_eg[:, D:], wg2))
    ef2 = ef0 + (h * gate) * _bdot(rbf, wer)
    pre_ng = p[:, 128:] + _bdot(ef2, wef_ng) + bng
    a_ng = _swish(pre_ng)
    h2 = _bdot(a_ng[:, :D], wn2)
    g2 = jax.nn.sigmoid(_bdot(a_ng[:, D:], wgn2))
    msg = (h2 * g2) * _bdot(rbf, wnr)
    return ef2, msg


def _tc_block(vivj, ef, rbf, wvi, wvj, wef_eg, wef_ng, beg, bng,
              we2, wg2, wn2, wgn2, wer_pad, wnr_pad):
    g_steps = E // TE

    def body(vi_ref, vj_ref, ef_ref, rbf_ref, wvi_ref, wvj_ref,
             wef_eg_ref, wef_ng_ref, beg_ref, bng_ref,
             we2_ref, wg2_ref, wn2_ref, wgn2_ref, wer_ref, wnr_ref,
             ef2_ref, msg_ref):
        ef2, msg = _block_math(
            vi_ref[...], vj_ref[...], ef_ref[...], rbf_ref[...],
            wvi_ref[...], wvj_ref[...], wef_eg_ref[...], wef_ng_ref[...],
            beg_ref[...], bng_ref[...], we2_ref[...], wg2_ref[...],
            wn2_ref[...], wgn2_ref[...], wer_ref[...], wnr_ref[...])
        ef2_ref[...] = ef2
        msg_ref[...] = msg

    full = lambda shape: pl.BlockSpec(shape, lambda g: (0, 0))
    return pl.pallas_call(
        body,
        grid=(g_steps,),
        in_specs=[
            pl.BlockSpec((TE, D), lambda g: (g, 0)),
            pl.BlockSpec((TE, D), lambda g, _gs=g_steps: (g + _gs, 0)),
            pl.BlockSpec((TE, D), lambda g: (g, 0)),
            pl.BlockSpec((TE, 16), lambda g: (g, 0)),
            full((D, 256)), full((D, 256)),
            full((D, 128)), full((D, 128)),
            full((1, 128)), full((1, 128)),
            full((D, D)), full((D, D)), full((D, D)), full((D, D)),
            full((16, D)), full((16, D)),
        ],
        out_specs=[
            pl.BlockSpec((TE, D), lambda g: (g, 0)),
            pl.BlockSpec((TE, D), lambda g: (g, 0)),
        ],
        out_shape=[
            jax.ShapeDtypeStruct((E, D), jnp.float32),
            jax.ShapeDtypeStruct((E, D), jnp.float32),
        ],
    )(vivj, vivj, ef, rbf, wvi, wvj, wef_eg, wef_ng, beg, bng,
      we2, wg2, wn2, wgn2, wer_pad, wnr_pad)


def _tc_block0(vivj, p2, wee_pad, bee2, wvi, wvj, wef_eg, wef_ng, beg, bng,
               we2, wg2, wn2, wgn2, wer_pad, wnr_pad):
    g_steps = E // TE

    def body(vi_ref, vj_ref, ps_ref, pd_ref, wee_ref, bee_ref,
             wvi_ref, wvj_ref, wef_eg_ref, wef_ng_ref, beg_ref, bng_ref,
             we2_ref, wg2_ref, wn2_ref, wgn2_ref, wer_ref, wnr_ref,
             ef2_ref, msg_ref, rbf_ref):
        rbf, ef0 = _rbf_ef(ps_ref[...], pd_ref[...], wee_ref[...],
                           bee_ref[...])
        rbf_ref[...] = rbf
        ef2, msg = _block_math(
            vi_ref[...], vj_ref[...], ef0, rbf,
            wvi_ref[...], wvj_ref[...], wef_eg_ref[...], wef_ng_ref[...],
            beg_ref[...], bng_ref[...], we2_ref[...], wg2_ref[...],
            wn2_ref[...], wgn2_ref[...], wer_ref[...], wnr_ref[...])
        ef2_ref[...] = ef2
        msg_ref[...] = msg

    full = lambda shape: pl.BlockSpec(shape, lambda g: (0, 0))
    return pl.pallas_call(
        body,
        grid=(g_steps,),
        in_specs=[
            pl.BlockSpec((TE, D), lambda g: (g, 0)),
            pl.BlockSpec((TE, D), lambda g, _gs=g_steps: (g + _gs, 0)),
            pl.BlockSpec((TE, 16), lambda g: (g, 0)),
            pl.BlockSpec((TE, 16), lambda g, _gs=g_steps: (g + _gs, 0)),
            full((16, D)), full((1, D)),
            full((D, 256)), full((D, 256)),
            full((D, 128)), full((D, 128)),
            full((1, 128)), full((1, 128)),
            full((D, D)), full((D, D)), full((D, D)), full((D, D)),
            full((16, D)), full((16, D)),
        ],
        out_specs=[
            pl.BlockSpec((TE, D), lambda g: (g, 0)),
            pl.BlockSpec((TE, D), lambda g: (g, 0)),
            pl.BlockSpec((TE, 16), lambda g: (g, 0)),
        ],
        out_shape=[
            jax.ShapeDtypeStruct((E, D), jnp.float32),
            jax.ShapeDtypeStruct((E, D), jnp.float32),
            jax.ShapeDtypeStruct((E, 16), jnp.float32),
        ],
    )(vivj, vivj, p2, p2, wee_pad, bee2, wvi, wvj, wef_eg, wef_ng,
      beg, bng, we2, wg2, wn2, wgn2, wer_pad, wnr_pad)


def _tc_readout(nf, wf1, bf1, wf2, bf2, wf3, bf3):
    g_steps = N // TN

    def body(nf_ref, wf1_ref, bf1_ref, wf2_ref, bf2_ref, wf3_ref, bf3_ref,
             out_ref, acc_ref):
        g = pl.program_id(0)

        @pl.when(g == 0)
        def _():
            acc_ref[...] = jnp.zeros_like(acc_ref)

        acc_ref[...] += jnp.sum(nf_ref[...], axis=0, keepdims=True)

        @pl.when(g == g_steps - 1)
        def _():
            pooled = acc_ref[...] * (1.0 / N)
            z = _swish(jnp.dot(pooled, wf1_ref[...],
                               preferred_element_type=jnp.float32)
                       + bf1_ref[...])
            z = _swish(jnp.dot(z, wf2_ref[...],
                               preferred_element_type=jnp.float32)
                       + bf2_ref[...])
            out_ref[...] = jnp.dot(z, wf3_ref[...],
                                   preferred_element_type=jnp.float32
                                   ) + bf3_ref[...]

    full = lambda shape: pl.BlockSpec(shape, lambda g: (0, 0))
    return pl.pallas_call(
        body,
        grid=(g_steps,),
        in_specs=[
            pl.BlockSpec((TN, D), lambda g: (g, 0)),
            full((D, D)), full((1, D)), full((D, D)), full((1, D)),
            full((D, 1)), full((1, 1)),
        ],
        out_specs=pl.BlockSpec((1, 1), lambda g: (0, 0)),
        out_shape=jax.ShapeDtypeStruct((1, 1), jnp.float32),
        scratch_shapes=[pltpu.VMEM((1, D), jnp.float32)],
    )(nf, wf1, bf1, wf2, bf2, wf3, bf3)


def kernel(node_type, pos, edge_index, node_embed, Wee, bee, We1, be1,
           Wg1, bg1, We2, Wg2, Wer, Wn1, bn1, Wgn1, bgn1, Wn2, Wgn2, Wnr,
           Wf1, bf1, Wf2, bf2, Wf3, bf3):
    idx2 = edge_index.astype(jnp.int32).reshape(2 * E)
    dst = edge_index[1].astype(jnp.int32)

    pos_pad = jnp.pad(pos, ((0, 0), (0, 13)))
    nt_pad = jnp.pad(node_type.astype(jnp.int32), (0, N_PAD - N))

    p2, nf = _sc_gather2(pos_pad, idx2, node_embed, nt_pad)

    wee_pad = jnp.pad(Wee, ((0, 16 - DEG), (0, 0)))
    rbf = None
    ef = None
    for b in range(NBLOCKS):
        vivj = _sc_gather(nf, idx2, D)
        w1 = jnp.concatenate(
            [We1[b], Wg1[b], Wn1[b], Wgn1[b]], axis=1)
        wvi, wvj, wef = w1[:D], w1[D:2 * D], w1[2 * D:]
        beg = jnp.concatenate([be1[b], bg1[b]]).reshape(1, 128)
        bng = jnp.concatenate([bn1[b], bgn1[b]]).reshape(1, 128)
        wer_pad = jnp.pad(Wer[b], ((0, 16 - DEG), (0, 0)))
        wnr_pad = jnp.pad(Wnr[b], ((0, 16 - DEG), (0, 0)))
        if b == 0:
            ef, msg, rbf = _tc_block0(
                vivj, p2, wee_pad, bee.reshape(1, D), wvi, wvj,
                wef[:, :128], wef[:, 128:], beg, bng,
                We2[b], Wg2[b], Wn2[b], Wgn2[b], wer_pad, wnr_pad)
        else:
            ef, msg = _tc_block(
                vivj, ef, rbf, wvi, wvj, wef[:, :128], wef[:, 128:],
                beg, bng, We2[b], Wg2[b], Wn2[b], Wgn2[b],
                wer_pad, wnr_pad)
        nf = _sc_scatter_add(msg, dst, nf)

    out = _tc_readout(nf, Wf1, bf1.reshape(1, D), Wf2, bf2.reshape(1, D),
                      Wf3, bf3.reshape(1, 1))
    return out.reshape(1)


# speedup vs baseline: 3.7871x; 1.3947x over previous
import functools

import jax
import jax.numpy as jnp
from jax import lax
from jax.experimental import pallas as pl
from jax.experimental.pallas import tpu as pltpu
from jax.experimental.pallas import tpu_sc as plsc

N = 50000
E = 800000
D = 64
NBLOCKS = 3
CUTOFF = 5.0
DEG = 9

N_PAD = 50048
CHUNK = 128
NW = 32
HALF = D // 2
TE = 3200
TN = 400

_mesh = functools.partial(
    plsc.VectorSubcoreMesh, core_axis_name="c", subcore_axis_name="s")


KBUF = 8


def _gather_loop(table_hbm, idx_hbm, out_hbm, idx_v, rows_v, sem, nchunks,
                 wid):
    iters = (nchunks + NW * KBUF - 1) // (NW * KBUF)

    def body(t, carry):
        def each(fn):
            for kk in range(KBUF):
                j = (t * KBUF + kk) * NW + wid

                @pl.when(j < nchunks)
                def _(j=j, kk=kk):
                    fn(j, kk)

        each(lambda j, kk: pltpu.async_copy(
            idx_hbm.at[pl.ds(j * CHUNK, CHUNK)], idx_v.at[kk], sem))
        each(lambda j, kk: pltpu.make_async_copy(
            idx_hbm.at[pl.ds(j * CHUNK, CHUNK)], idx_v.at[kk],
            sem).wait())
        each(lambda j, kk: pltpu.async_copy(
            table_hbm.at[idx_v.at[kk]], rows_v.at[kk], sem))
        each(lambda j, kk: pltpu.make_async_copy(
            table_hbm.at[idx_v.at[kk]], rows_v.at[kk], sem).wait())
        each(lambda j, kk: pltpu.async_copy(
            rows_v.at[kk], out_hbm.at[pl.ds(j * CHUNK, CHUNK)], sem))
        each(lambda j, kk: pltpu.make_async_copy(
            rows_v.at[kk], out_hbm.at[pl.ds(j * CHUNK, CHUNK)],
            sem).wait())
        return carry

    lax.fori_loop(0, iters, body, 0)


def _sc_gather(table, idx, d_t):
    m = idx.shape[0]
    nchunks = m // CHUNK

    @functools.partial(
        pl.kernel,
        mesh=_mesh(),
        compiler_params=pltpu.CompilerParams(use_tc_tiling_on_sc=False),
        out_type=jax.ShapeDtypeStruct((m, d_t), jnp.float32),
        scratch_types=[
            pltpu.VMEM((KBUF, CHUNK), jnp.int32),
            pltpu.VMEM((KBUF, CHUNK, d_t), jnp.float32),
            pltpu.SemaphoreType.DMA,
        ],
    )
    def k(table_hbm, idx_hbm, out_hbm, idx_v, rows_v, sem):
        wid = lax.axis_index("s") * 2 + lax.axis_index("c")
        _gather_loop(table_hbm, idx_hbm, out_hbm, idx_v, rows_v, sem,
                     nchunks, wid)

    return k(table, idx)


def _sc_gather2(pos_pad, idx2, node_embed, nt_pad):
    @functools.partial(
        pl.kernel,
        mesh=_mesh(),
        compiler_params=pltpu.CompilerParams(use_tc_tiling_on_sc=False),
        out_type=[
            jax.ShapeDtypeStruct((2 * E, 16), jnp.float32),
            jax.ShapeDtypeStruct((N_PAD, D), jnp.float32),
        ],
        scratch_types=[
            pltpu.VMEM((KBUF, CHUNK), jnp.int32),
            pltpu.VMEM((KBUF, CHUNK, 16), jnp.float32),
            pltpu.VMEM((KBUF, CHUNK, D), jnp.float32),
            pltpu.SemaphoreType.DMA,
        ],
    )
    def k(pos_hbm, idx2_hbm, emb_hbm, nt_hbm, p2_out, nf_out,
          idx_v, rows16_v, rows64_v, sem):
        wid = lax.axis_index("s") * 2 + lax.axis_index("c")
        _gather_loop(pos_hbm, idx2_hbm, p2_out, idx_v, rows16_v, sem,
                     (2 * E) // CHUNK, wid)
        _gather_loop(emb_hbm, nt_hbm, nf_out, idx_v, rows64_v, sem,
                     N_PAD // CHUNK, wid)

    return k(pos_pad, idx2, node_embed, nt_pad)


def _sc_scatter_add(msg, dst, node_feat):
    kb = 4
    nchunks = E // CHUNK
    iters = (nchunks + 16 * kb - 1) // (16 * kb)
    rows_per_tile = N_PAD // 16

    @functools.partial(
        pl.kernel,
        mesh=_mesh(),
        compiler_params=pltpu.CompilerParams(use_tc_tiling_on_sc=False),
        out_type=jax.ShapeDtypeStruct((N_PAD, D), jnp.float32),
        scratch_types=[
            pltpu.VMEM((kb, CHUNK), jnp.int32),
            pltpu.VMEM((kb, CHUNK, HALF), jnp.float32),
            pltpu.VMEM_SHARED((N_PAD, HALF), jnp.float32),
            pltpu.SemaphoreType.DMA,
        ],
    )
    def k(msg_hbm, dst_hbm, nf_hbm, out_hbm, idx_v, msg_v, acc_sh, sem):
        c = lax.axis_index("c")
        s = lax.axis_index("s")
        col0 = c * HALF
        row0 = s * rows_per_tile
        pltpu.sync_copy(
            nf_hbm.at[pl.ds(row0, rows_per_tile), pl.ds(col0, HALF)],
            acc_sh.at[pl.ds(row0, rows_per_tile)])
        plsc.subcore_barrier()

        def body(t, carry):
            def each(fn):
                for kk in range(kb):
                    j = (t * kb + kk) * 16 + s

                    @pl.when(j < nchunks)
                    def _(j=j, kk=kk):
                        fn(j, kk)

            def loads(j, kk):
                off = j * CHUNK
                pltpu.async_copy(
                    dst_hbm.at[pl.ds(off, CHUNK)], idx_v.at[kk], sem)
                pltpu.async_copy(
                    msg_hbm.at[pl.ds(off, CHUNK), pl.ds(D + col0, HALF)],
                    msg_v.at[kk], sem)

            def load_waits(j, kk):
                off = j * CHUNK
                pltpu.make_async_copy(
                    dst_hbm.at[pl.ds(off, CHUNK)], idx_v.at[kk], sem).wait()
                pltpu.make_async_copy(
                    msg_hbm.at[pl.ds(off, CHUNK), pl.ds(D + col0, HALF)],
                    msg_v.at[kk], sem).wait()

            each(loads)
            each(load_waits)
            each(lambda j, kk: pltpu.sync_copy(
                msg_v.at[kk], acc_sh.at[idx_v.at[kk]], add=True))
            return carry

        lax.fori_loop(0, iters, body, 0)
        plsc.subcore_barrier()
        pltpu.sync_copy(
            acc_sh.at[pl.ds(row0, rows_per_tile)],
            out_hbm.at[pl.ds(row0, rows_per_tile), pl.ds(col0, HALF)])

    return k(msg, dst, node_feat)


def _swish(x):
    return x * jax.nn.sigmoid(x)


def _bdot(a, b):
    return jnp.dot(a.astype(jnp.bfloat16), b.astype(jnp.bfloat16),
                   preferred_element_type=jnp.float32)


def _rbf_ef(ps, pd, wee, bee):
    dvec = pd - ps
    d2 = jnp.sum(dvec * dvec, axis=1, keepdims=True)
    bond = jnp.sqrt(d2 + 1e-12)
    r = jnp.maximum(bond, 1e-6)
    col = lax.broadcasted_iota(jnp.int32, (TE, 16), 1).astype(jnp.float32)
    nvec = col + 1.0
    t = nvec * (jnp.pi / CUTOFF) * jnp.minimum(r, CUTOFF)
    k = jnp.floor(t * (1.0 / jnp.pi) + 0.5)
    u = t - k * jnp.pi
    sign = 1.0 - 2.0 * (k - 2.0 * jnp.floor(k * 0.5))
    u2 = u * u
    sin_t = sign * u * (1.0 + u2 * (-1.0 / 6.0 + u2 * (
        1.0 / 120.0 + u2 * (-1.0 / 5040.0 + u2 * (1.0 / 362880.0)))))
    rbf = jnp.sqrt(2.0 / CUTOFF) * sin_t / r
    ratio = jnp.clip(bond / CUTOFF, 0.0, 1.0)
    env = 1.0 - 6.0 * ratio**5 + 15.0 * ratio**4 - 10.0 * ratio**3
    rbf = jnp.where(col < float(DEG), rbf * env, 0.0)
    pre = jnp.dot(rbf, wee, preferred_element_type=jnp.float32) + bee
    return rbf, _swish(pre)


def _block_math(x, ef0, rbf, wvivj, wef_eg, wef_ng, beg, bng,
                we2, wg2, wn2, wgn2, wer, wnr):
    p = _bdot(x, wvivj)
    pre_eg = p[:, :128] + _bdot(ef0, wef_eg) + beg
    a_eg = _swish(pre_eg)
    h = _bdot(a_eg[:, :D], we2)
    gate = jax.nn.sigmoid(_bdot(a_eg[:, D:], wg2))
    ef2 = ef0 + (h * gate) * _bdot(rbf, wer)
    pre_ng = p[:, 128:] + _bdot(ef2, wef_ng) + bng
    a_ng = _swish(pre_ng)
    h2 = _bdot(a_ng[:, :D], wn2)
    g2 = jax.nn.sigmoid(_bdot(a_ng[:, D:], wgn2))
    msg = (h2 * g2) * _bdot(rbf, wnr)
    return jnp.concatenate([ef2, msg], axis=1)


def _tc_block(vivj, efmsg, rbf, wvivj, wef_eg, wef_ng, beg, bng,
              we2, wg2, wn2, wgn2, wer_pad, wnr_pad):
    g_steps = E // TE

    def body(x_ref, prev_ref, rbf_ref, wvivj_ref,
             wef_eg_ref, wef_ng_ref, beg_ref, bng_ref,
             we2_ref, wg2_ref, wn2_ref, wgn2_ref, wer_ref, wnr_ref,
             out_ref):
        out_ref[...] = _block_math(
            x_ref[...], prev_ref[:, :D], rbf_ref[...],
            wvivj_ref[...], wef_eg_ref[...], wef_ng_ref[...],
            beg_ref[...], bng_ref[...], we2_ref[...], wg2_ref[...],
            wn2_ref[...], wgn2_ref[...], wer_ref[...], wnr_ref[...])

    full = lambda shape: pl.BlockSpec(shape, lambda g: (0, 0))
    return pl.pallas_call(
        body,
        grid=(g_steps,),
        in_specs=[
            pl.BlockSpec((TE, 2 * D), lambda g: (g, 0)),
            pl.BlockSpec((TE, 2 * D), lambda g: (g, 0)),
            pl.BlockSpec((TE, 16), lambda g: (g, 0)),
            full((2 * D, 256)),
            full((D, 128)), full((D, 128)),
            full((1, 128)), full((1, 128)),
            full((D, D)), full((D, D)), full((D, D)), full((D, D)),
            full((16, D)), full((16, D)),
        ],
        out_specs=pl.BlockSpec((TE, 2 * D), lambda g: (g, 0)),
        out_shape=jax.ShapeDtypeStruct((E, 2 * D), jnp.float32),
    )(vivj, efmsg, rbf, wvivj, wef_eg, wef_ng, beg, bng,
      we2, wg2, wn2, wgn2, wer_pad, wnr_pad)


def _tc_block0(vivj, p2, wee_pad, bee2, wvivj, wef_eg, wef_ng, beg, bng,
               we2, wg2, wn2, wgn2, wer_pad, wnr_pad):
    g_steps = E // TE

    def body(x_ref, ps_ref, pd_ref, wee_ref, bee_ref,
             wvivj_ref, wef_eg_ref, wef_ng_ref, beg_ref, bng_ref,
             we2_ref, wg2_ref, wn2_ref, wgn2_ref, wer_ref, wnr_ref,
             out_ref, rbf_ref):
        rbf, ef0 = _rbf_ef(ps_ref[...], pd_ref[...], wee_ref[...],
                           bee_ref[...])
        rbf_ref[...] = rbf
        out_ref[...] = _block_math(
            x_ref[...], ef0, rbf,
            wvivj_ref[...], wef_eg_ref[...], wef_ng_ref[...],
            beg_ref[...], bng_ref[...], we2_ref[...], wg2_ref[...],
            wn2_ref[...], wgn2_ref[...], wer_ref[...], wnr_ref[...])

    full = lambda shape: pl.BlockSpec(shape, lambda g: (0, 0))
    return pl.pallas_call(
        body,
        grid=(g_steps,),
        in_specs=[
            pl.BlockSpec((TE, 2 * D), lambda g: (g, 0)),
            pl.BlockSpec((TE, 16), lambda g: (g, 0)),
            pl.BlockSpec((TE, 16), lambda g, _gs=g_steps: (g + _gs, 0)),
            full((16, D)), full((1, D)),
            full((2 * D, 256)),
            full((D, 128)), full((D, 128)),
            full((1, 128)), full((1, 128)),
            full((D, D)), full((D, D)), full((D, D)), full((D, D)),
            full((16, D)), full((16, D)),
        ],
        out_specs=[
            pl.BlockSpec((TE, 2 * D), lambda g: (g, 0)),
            pl.BlockSpec((TE, 16), lambda g: (g, 0)),
        ],
        out_shape=[
            jax.ShapeDtypeStruct((E, 2 * D), jnp.float32),
            jax.ShapeDtypeStruct((E, 16), jnp.float32),
        ],
    )(vivj, p2, p2, wee_pad, bee2, wvivj, wef_eg, wef_ng,
      beg, bng, we2, wg2, wn2, wgn2, wer_pad, wnr_pad)


def _tc_readout(nf, wf1, bf1, wf2, bf2, wf3, bf3):
    g_steps = N // TN

    def body(nf_ref, wf1_ref, bf1_ref, wf2_ref, bf2_ref, wf3_ref, bf3_ref,
             out_ref, acc_ref):
        g = pl.program_id(0)

        @pl.when(g == 0)
        def _():
            acc_ref[...] = jnp.zeros_like(acc_ref)

        acc_ref[...] += jnp.sum(nf_ref[...], axis=0, keepdims=True)

        @pl.when(g == g_steps - 1)
        def _():
            pooled = acc_ref[...] * (1.0 / N)
            z = _swish(jnp.dot(pooled, wf1_ref[...],
                               preferred_element_type=jnp.float32)
                       + bf1_ref[...])
            z = _swish(jnp.dot(z, wf2_ref[...],
                               preferred_element_type=jnp.float32)
                       + bf2_ref[...])
            out_ref[...] = jnp.dot(z, wf3_ref[...],
                                   preferred_element_type=jnp.float32
                                   ) + bf3_ref[...]

    full = lambda shape: pl.BlockSpec(shape, lambda g: (0, 0))
    return pl.pallas_call(
        body,
        grid=(g_steps,),
        in_specs=[
            pl.BlockSpec((TN, D), lambda g: (g, 0)),
            full((D, D)), full((1, D)), full((D, D)), full((1, D)),
            full((D, 1)), full((1, 1)),
        ],
        out_specs=pl.BlockSpec((1, 1), lambda g: (0, 0)),
        out_shape=jax.ShapeDtypeStruct((1, 1), jnp.float32),
        scratch_shapes=[pltpu.VMEM((1, D), jnp.float32)],
    )(nf, wf1, bf1, wf2, bf2, wf3, bf3)


def kernel(node_type, pos, edge_index, node_embed, Wee, bee, We1, be1,
           Wg1, bg1, We2, Wg2, Wer, Wn1, bn1, Wgn1, bgn1, Wn2, Wgn2, Wnr,
           Wf1, bf1, Wf2, bf2, Wf3, bf3):
    idx2 = edge_index.astype(jnp.int32).reshape(2 * E)
    idx_int = edge_index.astype(jnp.int32).T.reshape(2 * E)
    dst = edge_index[1].astype(jnp.int32)

    pos_pad = jnp.pad(pos, ((0, 0), (0, 13)))
    nt_pad = jnp.pad(node_type.astype(jnp.int32), (0, N_PAD - N))

    p2, nf = _sc_gather2(pos_pad, idx2, node_embed, nt_pad)

    wee_pad = jnp.pad(Wee, ((0, 16 - DEG), (0, 0)))
    rbf = None
    ef = None
    for b in range(NBLOCKS):
        vivj = _sc_gather(nf, idx_int, D).reshape(E, 2 * D)
        w1 = jnp.concatenate(
            [We1[b], Wg1[b], Wn1[b], Wgn1[b]], axis=1)
        wvivj, wef = w1[:2 * D], w1[2 * D:]
        beg = jnp.concatenate([be1[b], bg1[b]]).reshape(1, 128)
        bng = jnp.concatenate([bn1[b], bgn1[b]]).reshape(1, 128)
        wer_pad = jnp.pad(Wer[b], ((0, 16 - DEG), (0, 0)))
        wnr_pad = jnp.pad(Wnr[b], ((0, 16 - DEG), (0, 0)))
        if b == 0:
            efmsg, rbf = _tc_block0(
                vivj, p2, wee_pad, bee.reshape(1, D), wvivj,
                wef[:, :128], wef[:, 128:], beg, bng,
                We2[b], Wg2[b], Wn2[b], Wgn2[b], wer_pad, wnr_pad)
        else:
            efmsg = _tc_block(
                vivj, efmsg, rbf, wvivj, wef[:, :128], wef[:, 128:],
                beg, bng, We2[b], Wg2[b], Wn2[b], Wgn2[b],
                wer_pad, wnr_pad)
        nf = _sc_scatter_add(efmsg, dst, nf)

    out = _tc_readout(nf, Wf1, bf1.reshape(1, D), Wf2, bf2.reshape(1, D),
                      Wf3, bf3.reshape(1, 1))
    return out.reshape(1)
